# Initial kernel scaffold; baseline (speedup 1.0000x reference)
import jax
import jax.numpy as jnp
from jax.experimental import pallas as pl


def kernel(chem_feats, nbr_vids, geom_feats, num_verts, eigs, params):
    raise NotImplementedError("write your pallas kernel here")


# speedup vs baseline: 1.8651x; 1.8651x over previous
import functools

import jax
import jax.numpy as jnp
from jax import lax
from jax.experimental import pallas as pl
from jax.experimental.pallas import tpu as pltpu
from jax.experimental.pallas import tpu_sc as plsc

_B = 8
_V = 2048
_N = _B * _V
_E = 262144
_K = 128
_H = 128
_CHEM = 34
_GEOM = 48
_EPS = 1e-5
_TE = 2048
_F32 = jnp.float32

_NSUB = 16
_HALF = _N // 2
_QUART = _N // 4
_CHUNK = 512
_NCHUNK = _E // _CHUNK
_CPS = _NCHUNK // _NSUB
_ACC_ROWS = 4224
_ZROWS = _ACC_ROWS // _NSUB
_DUMMY = _QUART


def _sigmoid(x):
    return 1.0 / (1.0 + jnp.exp(-x))


def _silu(x):
    return x * _sigmoid(x)


def _softplus(x):
    return jnp.maximum(x, 0.0) + jnp.log(1.0 + jnp.exp(-jnp.abs(x)))


def _bn(x, g, b):
    m = jnp.mean(x, axis=0, keepdims=True)
    v = jnp.mean((x - m) * (x - m), axis=0, keepdims=True)
    return g * (x - m) / jnp.sqrt(v + _EPS) + b


def _dot(a, b):
    return jnp.dot(a, b, preferred_element_type=_F32)


_VMEM_FULL = pltpu.CompilerParams(vmem_limit_bytes=64 * 1024 * 1024)


def _p1_stats(x, w1, b1):
    def body(x_ref, w_ref, b_ref, s_ref, q_ref):
        @pl.when(pl.program_id(0) == 0)
        def _():
            s_ref[...] = jnp.zeros_like(s_ref)
            q_ref[...] = jnp.zeros_like(q_ref)
        y = _dot(x_ref[...], w_ref[...]) + b_ref[...]
        s_ref[...] += jnp.sum(y, axis=0, keepdims=True)
        q_ref[...] += jnp.sum(y * y, axis=0, keepdims=True)

    return pl.pallas_call(
        body,
        grid=(_E // _TE,),
        in_specs=[pl.BlockSpec((_TE, _CHEM), lambda i: (i, 0)),
                  pl.BlockSpec((_CHEM, _H), lambda i: (0, 0)),
                  pl.BlockSpec((1, _H), lambda i: (0, 0))],
        out_specs=[pl.BlockSpec((1, _H), lambda i: (0, 0)),
                   pl.BlockSpec((1, _H), lambda i: (0, 0))],
        out_shape=[jax.ShapeDtypeStruct((1, _H), _F32)] * 2,
    )(x, w1, b1)


def _p2_stats(x, w1, b1, m1, g1, be1, w2, b2):
    def body(x_ref, w1_ref, b1_ref, m1_ref, g1_ref, be1_ref, w2_ref, b2_ref,
             s_ref, q_ref):
        @pl.when(pl.program_id(0) == 0)
        def _():
            s_ref[...] = jnp.zeros_like(s_ref)
            q_ref[...] = jnp.zeros_like(q_ref)
        y1 = _dot(x_ref[...], w1_ref[...]) + b1_ref[...]
        z = _silu((y1 - m1_ref[...]) * g1_ref[...] + be1_ref[...])
        y = _dot(z, w2_ref[...]) + b2_ref[...]
        s_ref[...] += jnp.sum(y, axis=0, keepdims=True)
        q_ref[...] += jnp.sum(y * y, axis=0, keepdims=True)

    return pl.pallas_call(
        body,
        grid=(_E // _TE,),
        in_specs=[pl.BlockSpec((_TE, _CHEM), lambda i: (i, 0)),
                  pl.BlockSpec((_CHEM, _H), lambda i: (0, 0)),
                  pl.BlockSpec((1, _H), lambda i: (0, 0)),
                  pl.BlockSpec((1, _H), lambda i: (0, 0)),
                  pl.BlockSpec((1, _H), lambda i: (0, 0)),
                  pl.BlockSpec((1, _H), lambda i: (0, 0)),
                  pl.BlockSpec((_H, 2 * _H), lambda i: (0, 0)),
                  pl.BlockSpec((1, 2 * _H), lambda i: (0, 0))],
        out_specs=[pl.BlockSpec((1, 2 * _H), lambda i: (0, 0)),
                   pl.BlockSpec((1, 2 * _H), lambda i: (0, 0))],
        out_shape=[jax.ShapeDtypeStruct((1, 2 * _H), _F32)] * 2,
    )(x, w1, b1, m1, g1, be1, w2, b2)


def _p3_gate(x, w1, b1, m1, g1, be1, w2, b2, m2, g2, be2):
    def body(x_ref, w1_ref, b1_ref, m1_ref, g1_ref, be1_ref, w2_ref, b2_ref,
             m2_ref, g2_ref, be2_ref, o_ref):
        y1 = _dot(x_ref[...], w1_ref[...]) + b1_ref[...]
        z = _silu((y1 - m1_ref[...]) * g1_ref[...] + be1_ref[...])
        y = _dot(z, w2_ref[...]) + b2_ref[...]
        y = (y - m2_ref[...]) * g2_ref[...] + be2_ref[...]
        u = y[:, :_H]
        v = y[:, _H:]
        o_ref[...] = _sigmoid(u) * _softplus(v)

    return pl.pallas_call(
        body,
        grid=(_E // _TE,),
        in_specs=[pl.BlockSpec((_TE, _CHEM), lambda i: (i, 0)),
                  pl.BlockSpec((_CHEM, _H), lambda i: (0, 0)),
                  pl.BlockSpec((1, _H), lambda i: (0, 0)),
                  pl.BlockSpec((1, _H), lambda i: (0, 0)),
                  pl.BlockSpec((1, _H), lambda i: (0, 0)),
                  pl.BlockSpec((1, _H), lambda i: (0, 0)),
                  pl.BlockSpec((_H, 2 * _H), lambda i: (0, 0)),
                  pl.BlockSpec((1, 2 * _H), lambda i: (0, 0)),
                  pl.BlockSpec((1, 2 * _H), lambda i: (0, 0)),
                  pl.BlockSpec((1, 2 * _H), lambda i: (0, 0)),
                  pl.BlockSpec((1, 2 * _H), lambda i: (0, 0))],
        out_specs=pl.BlockSpec((_TE, _H), lambda i: (i, 0)),
        out_shape=jax.ShapeDtypeStruct((_E, _H), _F32),
    )(x, w1, b1, m1, g1, be1, w2, b2, m2, g2, be2)


def _bn_stats(s, q, g):
    m = s / float(_E)
    var = q / float(_E) - m * m
    return m, g / jnp.sqrt(var + _EPS)


def _segment_sum_sc(gated, ids):
    zeros = jnp.zeros((_ACC_ROWS, _H), _F32)
    mesh = plsc.VectorSubcoreMesh(core_axis_name="c", subcore_axis_name="s")

    @functools.partial(
        pl.kernel,
        out_type=jax.ShapeDtypeStruct((_N, _H), _F32),
        mesh=mesh,
        compiler_params=pltpu.CompilerParams(needs_layout_passes=False),
        scratch_types=[
            pltpu.VMEM((_CHUNK,), jnp.int32),
            pltpu.VMEM((128,), jnp.int32),
            pltpu.VMEM((128,), jnp.int32),
            pltpu.VMEM((128,), jnp.int32),
            pltpu.VMEM((128,), jnp.int32),
            pltpu.VMEM((_CHUNK, _H), _F32),
            pltpu.VMEM_SHARED((_ACC_ROWS, _H), _F32),
        ],
    )
    def seg(vals_hbm, ids_hbm, zeros_hbm, out_hbm,
            ids_v, ix0, ix1, ix2, ix3, vals_v, acc):
        c = lax.axis_index("c")
        s = lax.axis_index("s")
        ix_refs = (ix0, ix1, ix2, ix3)
        rows = _QUART // _NSUB

        for p in range(2):
            base = c * _HALF + p * _QUART
            pltpu.sync_copy(zeros_hbm.at[pl.ds(s * _ZROWS, _ZROWS)],
                            acc.at[pl.ds(s * _ZROWS, _ZROWS)])
            plsc.subcore_barrier()

            @pl.loop(0, _CPS)
            def _(j):
                chunk = j * _NSUB + s
                estart = chunk * _CHUNK
                pltpu.sync_copy(ids_hbm.at[pl.ds(estart, _CHUNK)], ids_v)
                lo = jnp.min(ids_v[pl.ds(0, 16)])
                hi = jnp.max(ids_v[pl.ds(_CHUNK - 16, 16)])
                hit = jnp.logical_and(hi >= base, lo < base + _QUART)

                @pl.when(hit)
                def _():
                    pltpu.sync_copy(vals_hbm.at[pl.ds(estart, _CHUNK)],
                                    vals_v)
                    for r in range(4):
                        for k in range(8):
                            sl = ids_v[pl.ds(r * 128 + k * 16, 16)]
                            rel = sl - base
                            ok = jnp.logical_and(rel >= 0, rel < _QUART)
                            ix_refs[r][pl.ds(k * 16, 16)] = jnp.where(
                                ok, rel, _DUMMY)
                    for r in range(4):
                        pltpu.sync_copy(vals_v.at[pl.ds(r * 128, 128)],
                                        acc.at[ix_refs[r]], add=True)

            plsc.subcore_barrier()
            pltpu.sync_copy(acc.at[pl.ds(s * rows, rows)],
                            out_hbm.at[pl.ds(base + s * rows, rows)])
            plsc.subcore_barrier()

    return seg(gated, ids, zeros)


def _p5a_geom(gf, gw1, gb1, gg1, gbb1, gw2, gb2, gg2, gbb2):
    def body(gf_ref, gw1_r, gb1_r, gg1_r, gbb1_r, gw2_r, gb2_r,
             gg2_r, gbb2_r, o_ref):
        y = _dot(gf_ref[...], gw1_r[...]) + gb1_r[...]
        y = _silu(_bn(y, gg1_r[...], gbb1_r[...]))
        o_ref[...] = _bn(_dot(y, gw2_r[...]) + gb2_r[...], gg2_r[...],
                         gbb2_r[...])

    return pl.pallas_call(
        body,
        compiler_params=_VMEM_FULL,
        out_shape=jax.ShapeDtypeStruct((_N, _H // 2), _F32),
    )(gf, gw1, gb1, gg1, gbb1, gw2, gb2, gg2, gbb2)


def _p5a_feat(h_chem, hg, fwa, fwb, fb1, fg1, fbb1, fw2, fb2, fg2, fbb2):
    def body(hc_ref, hg_ref, fwa_r, fwb_r, fb1_r, fg1_r, fbb1_r, fw2_r,
             fb2_r, fg2_r, fbb2_r, o_ref):
        y = _dot(hc_ref[...], fwa_r[...]) + _dot(hg_ref[...], fwb_r[...]) \
            + fb1_r[...]
        y = _silu(_bn(y, fg1_r[...], fbb1_r[...]))
        y = _dot(y, fw2_r[...]) + fb2_r[...]
        o_ref[...] = _bn(y, fg2_r[...], fbb2_r[...])

    return pl.pallas_call(
        body,
        compiler_params=_VMEM_FULL,
        out_shape=jax.ShapeDtypeStruct((_N, _H), _F32),
    )(h_chem, hg, fwa, fwb, fb1, fg1, fbb1, fw2, fb2, fg2, fbb2)


def _p5b_spectral(h, vecs, vinv_t, evb, t, m, sd):
    def body(h_ref, v_ref, vi_ref, e_ref, t_ref, m_ref, s_ref, o_ref):
        hs = _dot(vi_ref[0], h_ref[...])
        e = e_ref[0]
        d = m_ref[...] - e
        band = jnp.exp(-(d * d) / (2.0 * s_ref[...] * s_ref[...]))
        prop = jnp.exp(-e * t_ref[...])
        o_ref[...] = _dot(v_ref[0], band * prop * hs)

    return pl.pallas_call(
        body,
        grid=(_B,),
        in_specs=[pl.BlockSpec((_V, _H), lambda b: (b, 0)),
                  pl.BlockSpec((1, _V, _K), lambda b: (b, 0, 0)),
                  pl.BlockSpec((1, _K, _V), lambda b: (b, 0, 0)),
                  pl.BlockSpec((1, _K, _H), lambda b: (b, 0, 0)),
                  pl.BlockSpec((1, _H), lambda b: (0, 0)),
                  pl.BlockSpec((1, _H), lambda b: (0, 0)),
                  pl.BlockSpec((1, _H), lambda b: (0, 0))],
        out_specs=pl.BlockSpec((_V, _H), lambda b: (b, 0)),
        out_shape=jax.ShapeDtypeStruct((_N, _H), _F32),
        compiler_params=_VMEM_FULL,
    )(h, vecs, vinv_t, evb, t, m, sd)


def _p5c_update(h, hp, wa, wb, b1, g1, bb1, w2, b2, g2, bb2):
    def body(h_ref, hp_ref, wa_r, wb_r, b1_r, g1_r, bb1_r, w2_r, b2_r,
             g2_r, bb2_r, o_ref):
        y = _dot(h_ref[...], wa_r[...]) + _dot(hp_ref[...], wb_r[...]) \
            + b1_r[...]
        y = _silu(_bn(y, g1_r[...], bb1_r[...]))
        y = _dot(y, w2_r[...]) + b2_r[...]
        o_ref[...] = h_ref[...] + _bn(y, g2_r[...], bb2_r[...])

    return pl.pallas_call(
        body,
        compiler_params=_VMEM_FULL,
        out_shape=jax.ShapeDtypeStruct((_N, _H), _F32),
    )(h, hp, wa, wb, b1, g1, bb1, w2, b2, g2, bb2)


def _p5d_head(h, wc, bc, gc, bcb, ws, bs):
    def body(h_ref, wc_r, bc_r, gc_r, bcb_r, ws_r, bs_r, o_ref):
        hm = jnp.mean(h_ref[...].reshape(_B, _V, _H), axis=1)
        y = _dot(hm, wc_r[...]) + bc_r[...]
        y = _silu(_bn(y, gc_r[...], bcb_r[...]))
        o_ref[...] = _dot(y, ws_r[...]) + bs_r[...]

    return pl.pallas_call(
        body,
        compiler_params=_VMEM_FULL,
        out_shape=jax.ShapeDtypeStruct((_B, 7), _F32),
    )(h, wc, bc, gc, bcb, ws, bs)


def _row(x):
    return x.reshape(1, -1)


def kernel(chem_feats, nbr_vids, geom_feats, num_verts, eigs, params):
    del num_verts
    cm = params["chem_mlp"]
    w1, b1 = cm["l1"]["W"], _row(cm["l1"]["b"])
    w2, b2 = cm["l2"]["W"], _row(cm["l2"]["b"])

    s1, q1 = _p1_stats(chem_feats, w1, b1)
    m1, g1 = _bn_stats(s1, q1, _row(cm["bn1"]["g"]))
    be1 = _row(cm["bn1"]["b"])

    s2, q2 = _p2_stats(chem_feats, w1, b1, m1, g1, be1, w2, b2)
    m2, g2 = _bn_stats(s2, q2, _row(cm["bn2"]["g"]))
    be2 = _row(cm["bn2"]["b"])

    gated = _p3_gate(chem_feats, w1, b1, m1, g1, be1, w2, b2, m2, g2, be2)
    h_chem = _segment_sum_sc(gated, nbr_vids)

    gm, fm = params["geom_mlp"], params["feat_mlp"]
    fw1 = fm["l1"]["W"]
    hg = _p5a_geom(
        geom_feats,
        gm["l1"]["W"], _row(gm["l1"]["b"]), _row(gm["bn1"]["g"]),
        _row(gm["bn1"]["b"]), gm["l2"]["W"], _row(gm["l2"]["b"]),
        _row(gm["bn2"]["g"]), _row(gm["bn2"]["b"]))
    h = _p5a_feat(
        h_chem, hg,
        fw1[:_H], fw1[_H:], _row(fm["l1"]["b"]), _row(fm["bn1"]["g"]),
        _row(fm["bn1"]["b"]), fm["l2"]["W"], _row(fm["l2"]["b"]),
        _row(fm["bn2"]["g"]), _row(fm["bn2"]["b"]))

    ev = eigs[:, 0, :]
    vecs = eigs[:, 1:1 + _V, :]
    vinv_t = jnp.transpose(eigs[:, 1 + _V:, :], (0, 2, 1))
    evb = jnp.broadcast_to(ev[:, :, None], (_B, _K, _H))

    for lp in params["prop"]:
        t = _row(jnp.clip(lp["t"], 1e-6, None))
        mn = _row(jnp.clip(lp["mean"], 0.001, 5.0))
        sd = _row(jnp.clip(lp["std"], 1.0, None))
        hp = _p5b_spectral(h, vecs, vinv_t, evb, t, mn, sd)
        mp = lp["mlp"]
        mw1 = mp["l1"]["W"]
        h = _p5c_update(
            h, hp, mw1[:_H], mw1[_H:], _row(mp["l1"]["b"]),
            _row(mp["bn1"]["g"]), _row(mp["bn1"]["b"]), mp["l2"]["W"],
            _row(mp["l2"]["b"]), _row(mp["bn2"]["g"]), _row(mp["bn2"]["b"]))

    cl = params["clas"]
    return _p5d_head(h, cl["l"]["W"], _row(cl["l"]["b"]), _row(cl["bn"]["g"]),
                     _row(cl["bn"]["b"]), params["scorer"]["W"],
                     _row(params["scorer"]["b"]))


# speedup vs baseline: 2.2294x; 1.1953x over previous
import functools

import jax
import jax.numpy as jnp
from jax import lax
from jax.experimental import pallas as pl
from jax.experimental.pallas import tpu as pltpu
from jax.experimental.pallas import tpu_sc as plsc

_B = 8
_V = 2048
_N = _B * _V
_E = 262144
_K = 128
_H = 128
_CHEM = 34
_GEOM = 48
_EPS = 1e-5
_TE = 8192
_F32 = jnp.float32

_NSUB = 16
_HALF = _N // 2
_QUART = _N // 4
_CHUNK = 512
_NCHUNK = _E // _CHUNK
_CPS = _NCHUNK // _NSUB
_ACC_ROWS = 4224
_ZROWS = _ACC_ROWS // _NSUB
_DUMMY = _QUART


def _sigmoid(x):
    return 1.0 / (1.0 + jnp.exp(-x))


def _silu(x):
    return x * _sigmoid(x)


def _softplus(x):
    return jnp.maximum(x, 0.0) + jnp.log(1.0 + jnp.exp(-jnp.abs(x)))


def _bn(x, g, b):
    m = jnp.mean(x, axis=0, keepdims=True)
    v = jnp.mean((x - m) * (x - m), axis=0, keepdims=True)
    return g * (x - m) / jnp.sqrt(v + _EPS) + b


def _dot(a, b):
    return jnp.dot(a, b, preferred_element_type=_F32)


_VMEM_FULL = pltpu.CompilerParams(vmem_limit_bytes=64 * 1024 * 1024)


def _p1_stats(x, w1, b1):
    def body(x_ref, w_ref, b_ref, s_ref, q_ref):
        @pl.when(pl.program_id(0) == 0)
        def _():
            s_ref[...] = jnp.zeros_like(s_ref)
            q_ref[...] = jnp.zeros_like(q_ref)
        y = _dot(x_ref[...], w_ref[...]) + b_ref[...]
        s_ref[...] += jnp.sum(y, axis=0, keepdims=True)
        q_ref[...] += jnp.sum(y * y, axis=0, keepdims=True)

    return pl.pallas_call(
        body,
        grid=(_E // _TE,),
        in_specs=[pl.BlockSpec((_TE, _CHEM), lambda i: (i, 0)),
                  pl.BlockSpec((_CHEM, _H), lambda i: (0, 0)),
                  pl.BlockSpec((1, _H), lambda i: (0, 0))],
        out_specs=[pl.BlockSpec((1, _H), lambda i: (0, 0)),
                   pl.BlockSpec((1, _H), lambda i: (0, 0))],
        out_shape=[jax.ShapeDtypeStruct((1, _H), _F32)] * 2,
    )(x, w1, b1)


def _p2_stats(x, w1, b1, m1, g1, be1, w2, b2):
    def body(x_ref, w1_ref, b1_ref, m1_ref, g1_ref, be1_ref, w2_ref, b2_ref,
             s_ref, q_ref):
        @pl.when(pl.program_id(0) == 0)
        def _():
            s_ref[...] = jnp.zeros_like(s_ref)
            q_ref[...] = jnp.zeros_like(q_ref)
        y1 = _dot(x_ref[...], w1_ref[...]) + b1_ref[...]
        z = _silu((y1 - m1_ref[...]) * g1_ref[...] + be1_ref[...])
        y = _dot(z, w2_ref[...]) + b2_ref[...]
        s_ref[...] += jnp.sum(y, axis=0, keepdims=True)
        q_ref[...] += jnp.sum(y * y, axis=0, keepdims=True)

    return pl.pallas_call(
        body,
        grid=(_E // _TE,),
        in_specs=[pl.BlockSpec((_TE, _CHEM), lambda i: (i, 0)),
                  pl.BlockSpec((_CHEM, _H), lambda i: (0, 0)),
                  pl.BlockSpec((1, _H), lambda i: (0, 0)),
                  pl.BlockSpec((1, _H), lambda i: (0, 0)),
                  pl.BlockSpec((1, _H), lambda i: (0, 0)),
                  pl.BlockSpec((1, _H), lambda i: (0, 0)),
                  pl.BlockSpec((_H, 2 * _H), lambda i: (0, 0)),
                  pl.BlockSpec((1, 2 * _H), lambda i: (0, 0))],
        out_specs=[pl.BlockSpec((1, 2 * _H), lambda i: (0, 0)),
                   pl.BlockSpec((1, 2 * _H), lambda i: (0, 0))],
        out_shape=[jax.ShapeDtypeStruct((1, 2 * _H), _F32)] * 2,
    )(x, w1, b1, m1, g1, be1, w2, b2)


def _p3_gate(x, w1, b1, m1, g1, be1, w2, b2, m2, g2, be2):
    def body(x_ref, w1_ref, b1_ref, m1_ref, g1_ref, be1_ref, w2_ref, b2_ref,
             m2_ref, g2_ref, be2_ref, o_ref):
        y1 = _dot(x_ref[...], w1_ref[...]) + b1_ref[...]
        z = _silu((y1 - m1_ref[...]) * g1_ref[...] + be1_ref[...])
        y = _dot(z, w2_ref[...]) + b2_ref[...]
        y = (y - m2_ref[...]) * g2_ref[...] + be2_ref[...]
        u = y[:, :_H]
        v = y[:, _H:]
        o_ref[...] = _sigmoid(u) * _softplus(v)

    return pl.pallas_call(
        body,
        grid=(_E // _TE,),
        in_specs=[pl.BlockSpec((_TE, _CHEM), lambda i: (i, 0)),
                  pl.BlockSpec((_CHEM, _H), lambda i: (0, 0)),
                  pl.BlockSpec((1, _H), lambda i: (0, 0)),
                  pl.BlockSpec((1, _H), lambda i: (0, 0)),
                  pl.BlockSpec((1, _H), lambda i: (0, 0)),
                  pl.BlockSpec((1, _H), lambda i: (0, 0)),
                  pl.BlockSpec((_H, 2 * _H), lambda i: (0, 0)),
                  pl.BlockSpec((1, 2 * _H), lambda i: (0, 0)),
                  pl.BlockSpec((1, 2 * _H), lambda i: (0, 0)),
                  pl.BlockSpec((1, 2 * _H), lambda i: (0, 0)),
                  pl.BlockSpec((1, 2 * _H), lambda i: (0, 0))],
        out_specs=pl.BlockSpec((_TE, _H), lambda i: (i, 0)),
        out_shape=jax.ShapeDtypeStruct((_E, _H), _F32),
    )(x, w1, b1, m1, g1, be1, w2, b2, m2, g2, be2)


def _bn_stats(s, q, g):
    m = s / float(_E)
    var = q / float(_E) - m * m
    return m, g / jnp.sqrt(var + _EPS)


def _segment_sum_sc(gated, ids):
    zeros = jnp.zeros((_ACC_ROWS, _H), _F32)
    mesh = plsc.VectorSubcoreMesh(core_axis_name="c", subcore_axis_name="s")

    @functools.partial(
        pl.kernel,
        out_type=jax.ShapeDtypeStruct((_N, _H), _F32),
        mesh=mesh,
        compiler_params=pltpu.CompilerParams(needs_layout_passes=False),
        scratch_types=[
            pltpu.VMEM((_CHUNK,), jnp.int32),
            pltpu.VMEM((128,), jnp.int32),
            pltpu.VMEM((128,), jnp.int32),
            pltpu.VMEM((128,), jnp.int32),
            pltpu.VMEM((128,), jnp.int32),
            pltpu.VMEM((_CHUNK, _H), _F32),
            pltpu.VMEM_SHARED((_ACC_ROWS, _H), _F32),
        ],
    )
    def seg(vals_hbm, ids_hbm, zeros_hbm, out_hbm,
            ids_v, ix0, ix1, ix2, ix3, vals_v, acc):
        c = lax.axis_index("c")
        s = lax.axis_index("s")
        ix_refs = (ix0, ix1, ix2, ix3)
        rows = _QUART // _NSUB

        for p in range(2):
            base = c * _HALF + p * _QUART
            pltpu.sync_copy(zeros_hbm.at[pl.ds(s * _ZROWS, _ZROWS)],
                            acc.at[pl.ds(s * _ZROWS, _ZROWS)])
            plsc.subcore_barrier()

            @pl.loop(0, _CPS)
            def _(j):
                chunk = j * _NSUB + s
                estart = chunk * _CHUNK
                pltpu.sync_copy(ids_hbm.at[pl.ds(estart, _CHUNK)], ids_v)
                lo = jnp.min(ids_v[pl.ds(0, 16)])
                hi = jnp.max(ids_v[pl.ds(_CHUNK - 16, 16)])
                hit = jnp.logical_and(hi >= base, lo < base + _QUART)

                @pl.when(hit)
                def _():
                    pltpu.sync_copy(vals_hbm.at[pl.ds(estart, _CHUNK)],
                                    vals_v)
                    for r in range(4):
                        for k in range(8):
                            sl = ids_v[pl.ds(r * 128 + k * 16, 16)]
                            rel = sl - base
                            ok = jnp.logical_and(rel >= 0, rel < _QUART)
                            ix_refs[r][pl.ds(k * 16, 16)] = jnp.where(
                                ok, rel, _DUMMY)
                    for r in range(4):
                        pltpu.sync_copy(vals_v.at[pl.ds(r * 128, 128)],
                                        acc.at[ix_refs[r]], add=True)

            plsc.subcore_barrier()
            pltpu.sync_copy(acc.at[pl.ds(s * rows, rows)],
                            out_hbm.at[pl.ds(base + s * rows, rows)])
            plsc.subcore_barrier()

    return seg(gated, ids, zeros)


def _p5a_geom(gf, gw1, gb1, gg1, gbb1, gw2, gb2, gg2, gbb2):
    def body(gf_ref, gw1_r, gb1_r, gg1_r, gbb1_r, gw2_r, gb2_r,
             gg2_r, gbb2_r, o_ref):
        y = _dot(gf_ref[...], gw1_r[...]) + gb1_r[...]
        y = _silu(_bn(y, gg1_r[...], gbb1_r[...]))
        o_ref[...] = _bn(_dot(y, gw2_r[...]) + gb2_r[...], gg2_r[...],
                         gbb2_r[...])

    return pl.pallas_call(
        body,
        compiler_params=_VMEM_FULL,
        out_shape=jax.ShapeDtypeStruct((_N, _H // 2), _F32),
    )(gf, gw1, gb1, gg1, gbb1, gw2, gb2, gg2, gbb2)


def _p5a_feat(h_chem, hg, fwa, fwb, fb1, fg1, fbb1, fw2, fb2, fg2, fbb2):
    def body(hc_ref, hg_ref, fwa_r, fwb_r, fb1_r, fg1_r, fbb1_r, fw2_r,
             fb2_r, fg2_r, fbb2_r, o_ref):
        y = _dot(hc_ref[...], fwa_r[...]) + _dot(hg_ref[...], fwb_r[...]) \
            + fb1_r[...]
        y = _silu(_bn(y, fg1_r[...], fbb1_r[...]))
        y = _dot(y, fw2_r[...]) + fb2_r[...]
        o_ref[...] = _bn(y, fg2_r[...], fbb2_r[...])

    return pl.pallas_call(
        body,
        compiler_params=_VMEM_FULL,
        out_shape=jax.ShapeDtypeStruct((_N, _H), _F32),
    )(h_chem, hg, fwa, fwb, fb1, fg1, fbb1, fw2, fb2, fg2, fbb2)


def _p5b_spectral(h, vecs, vinv_t, evb, t, m, sd):
    def body(h_ref, v_ref, vi_ref, e_ref, t_ref, m_ref, s_ref, o_ref):
        hs = _dot(vi_ref[0], h_ref[...])
        e = e_ref[0]
        d = m_ref[...] - e
        band = jnp.exp(-(d * d) / (2.0 * s_ref[...] * s_ref[...]))
        prop = jnp.exp(-e * t_ref[...])
        o_ref[...] = _dot(v_ref[0], band * prop * hs)

    return pl.pallas_call(
        body,
        grid=(_B,),
        in_specs=[pl.BlockSpec((_V, _H), lambda b: (b, 0)),
                  pl.BlockSpec((1, _V, _K), lambda b: (b, 0, 0)),
                  pl.BlockSpec((1, _K, _V), lambda b: (b, 0, 0)),
                  pl.BlockSpec((1, _K, _H), lambda b: (b, 0, 0)),
                  pl.BlockSpec((1, _H), lambda b: (0, 0)),
                  pl.BlockSpec((1, _H), lambda b: (0, 0)),
                  pl.BlockSpec((1, _H), lambda b: (0, 0))],
        out_specs=pl.BlockSpec((_V, _H), lambda b: (b, 0)),
        out_shape=jax.ShapeDtypeStruct((_N, _H), _F32),
        compiler_params=_VMEM_FULL,
    )(h, vecs, vinv_t, evb, t, m, sd)


def _p5c_update(h, hp, wa, wb, b1, g1, bb1, w2, b2, g2, bb2):
    def body(h_ref, hp_ref, wa_r, wb_r, b1_r, g1_r, bb1_r, w2_r, b2_r,
             g2_r, bb2_r, o_ref):
        y = _dot(h_ref[...], wa_r[...]) + _dot(hp_ref[...], wb_r[...]) \
            + b1_r[...]
        y = _silu(_bn(y, g1_r[...], bb1_r[...]))
        y = _dot(y, w2_r[...]) + b2_r[...]
        o_ref[...] = h_ref[...] + _bn(y, g2_r[...], bb2_r[...])

    return pl.pallas_call(
        body,
        compiler_params=_VMEM_FULL,
        out_shape=jax.ShapeDtypeStruct((_N, _H), _F32),
    )(h, hp, wa, wb, b1, g1, bb1, w2, b2, g2, bb2)


def _p5d_head(h, wc, bc, gc, bcb, ws, bs):
    def body(h_ref, wc_r, bc_r, gc_r, bcb_r, ws_r, bs_r, o_ref):
        hm = jnp.mean(h_ref[...].reshape(_B, _V, _H), axis=1)
        y = _dot(hm, wc_r[...]) + bc_r[...]
        y = _silu(_bn(y, gc_r[...], bcb_r[...]))
        o_ref[...] = _dot(y, ws_r[...]) + bs_r[...]

    return pl.pallas_call(
        body,
        compiler_params=_VMEM_FULL,
        out_shape=jax.ShapeDtypeStruct((_B, 7), _F32),
    )(h, wc, bc, gc, bcb, ws, bs)


def _row(x):
    return x.reshape(1, -1)


def kernel(chem_feats, nbr_vids, geom_feats, num_verts, eigs, params):
    del num_verts
    cm = params["chem_mlp"]
    w1, b1 = cm["l1"]["W"], _row(cm["l1"]["b"])
    w2, b2 = cm["l2"]["W"], _row(cm["l2"]["b"])

    s1, q1 = _p1_stats(chem_feats, w1, b1)
    m1, g1 = _bn_stats(s1, q1, _row(cm["bn1"]["g"]))
    be1 = _row(cm["bn1"]["b"])

    s2, q2 = _p2_stats(chem_feats, w1, b1, m1, g1, be1, w2, b2)
    m2, g2 = _bn_stats(s2, q2, _row(cm["bn2"]["g"]))
    be2 = _row(cm["bn2"]["b"])

    gated = _p3_gate(chem_feats, w1, b1, m1, g1, be1, w2, b2, m2, g2, be2)
    h_chem = _segment_sum_sc(gated, nbr_vids)

    gm, fm = params["geom_mlp"], params["feat_mlp"]
    fw1 = fm["l1"]["W"]
    hg = _p5a_geom(
        geom_feats,
        gm["l1"]["W"], _row(gm["l1"]["b"]), _row(gm["bn1"]["g"]),
        _row(gm["bn1"]["b"]), gm["l2"]["W"], _row(gm["l2"]["b"]),
        _row(gm["bn2"]["g"]), _row(gm["bn2"]["b"]))
    h = _p5a_feat(
        h_chem, hg,
        fw1[:_H], fw1[_H:], _row(fm["l1"]["b"]), _row(fm["bn1"]["g"]),
        _row(fm["bn1"]["b"]), fm["l2"]["W"], _row(fm["l2"]["b"]),
        _row(fm["bn2"]["g"]), _row(fm["bn2"]["b"]))

    ev = eigs[:, 0, :]
    vecs = eigs[:, 1:1 + _V, :]
    vinv_t = jnp.transpose(eigs[:, 1 + _V:, :], (0, 2, 1))
    evb = jnp.broadcast_to(ev[:, :, None], (_B, _K, _H))

    for lp in params["prop"]:
        t = _row(jnp.clip(lp["t"], 1e-6, None))
        mn = _row(jnp.clip(lp["mean"], 0.001, 5.0))
        sd = _row(jnp.clip(lp["std"], 1.0, None))
        hp = _p5b_spectral(h, vecs, vinv_t, evb, t, mn, sd)
        mp = lp["mlp"]
        mw1 = mp["l1"]["W"]
        h = _p5c_update(
            h, hp, mw1[:_H], mw1[_H:], _row(mp["l1"]["b"]),
            _row(mp["bn1"]["g"]), _row(mp["bn1"]["b"]), mp["l2"]["W"],
            _row(mp["l2"]["b"]), _row(mp["bn2"]["g"]), _row(mp["bn2"]["b"]))

    cl = params["clas"]
    return _p5d_head(h, cl["l"]["W"], _row(cl["l"]["b"]), _row(cl["bn"]["g"]),
                     _row(cl["bn"]["b"]), params["scorer"]["W"],
                     _row(params["scorer"]["b"]))


# speedup vs baseline: 2.2765x; 1.0212x over previous
import functools

import jax
import jax.numpy as jnp
from jax import lax
from jax.experimental import pallas as pl
from jax.experimental.pallas import tpu as pltpu
from jax.experimental.pallas import tpu_sc as plsc

_B = 8
_V = 2048
_N = _B * _V
_E = 262144
_K = 128
_H = 128
_CHEM = 34
_GEOM = 48
_EPS = 1e-5
_TE = 8192
_F32 = jnp.float32

_NSUB = 16
_HALF = _N // 2
_QUART = _N // 4
_CHUNK = 512
_NCHUNK = _E // _CHUNK
_CPS = _NCHUNK // _NSUB
_ACC_ROWS = 4224
_ZROWS = _ACC_ROWS // _NSUB
_DUMMY = _QUART


def _sigmoid(x):
    return 1.0 / (1.0 + jnp.exp(-x))


def _silu(x):
    return x * _sigmoid(x)


def _softplus(x):
    return jnp.maximum(x, 0.0) + jnp.log(1.0 + jnp.exp(-jnp.abs(x)))


def _bn(x, g, b):
    m = jnp.mean(x, axis=0, keepdims=True)
    v = jnp.mean((x - m) * (x - m), axis=0, keepdims=True)
    return g * (x - m) / jnp.sqrt(v + _EPS) + b


def _dot(a, b):
    return jnp.dot(a, b, preferred_element_type=_F32)


_VMEM_FULL = pltpu.CompilerParams(vmem_limit_bytes=64 * 1024 * 1024)


def _p1_stats(x, w1, b1):
    def body(x_ref, w_ref, b_ref, s_ref, q_ref):
        @pl.when(pl.program_id(0) == 0)
        def _():
            s_ref[...] = jnp.zeros_like(s_ref)
            q_ref[...] = jnp.zeros_like(q_ref)
        y = _dot(x_ref[...], w_ref[...]) + b_ref[...]
        s_ref[...] += jnp.sum(y, axis=0, keepdims=True)
        q_ref[...] += jnp.sum(y * y, axis=0, keepdims=True)

    return pl.pallas_call(
        body,
        grid=(_E // _TE,),
        in_specs=[pl.BlockSpec((_TE, _CHEM), lambda i: (i, 0)),
                  pl.BlockSpec((_CHEM, _H), lambda i: (0, 0)),
                  pl.BlockSpec((1, _H), lambda i: (0, 0))],
        out_specs=[pl.BlockSpec((1, _H), lambda i: (0, 0)),
                   pl.BlockSpec((1, _H), lambda i: (0, 0))],
        out_shape=[jax.ShapeDtypeStruct((1, _H), _F32)] * 2,
    )(x, w1, b1)


def _p2_stats(x, w1, b1, m1, g1, be1, w2, b2):
    def body(x_ref, w1_ref, b1_ref, m1_ref, g1_ref, be1_ref, w2_ref, b2_ref,
             s_ref, q_ref):
        @pl.when(pl.program_id(0) == 0)
        def _():
            s_ref[...] = jnp.zeros_like(s_ref)
            q_ref[...] = jnp.zeros_like(q_ref)
        y1 = _dot(x_ref[...], w1_ref[...]) + b1_ref[...]
        z = _silu((y1 - m1_ref[...]) * g1_ref[...] + be1_ref[...])
        y = _dot(z, w2_ref[...]) + b2_ref[...]
        s_ref[...] += jnp.sum(y, axis=0, keepdims=True)
        q_ref[...] += jnp.sum(y * y, axis=0, keepdims=True)

    return pl.pallas_call(
        body,
        grid=(_E // _TE,),
        in_specs=[pl.BlockSpec((_TE, _CHEM), lambda i: (i, 0)),
                  pl.BlockSpec((_CHEM, _H), lambda i: (0, 0)),
                  pl.BlockSpec((1, _H), lambda i: (0, 0)),
                  pl.BlockSpec((1, _H), lambda i: (0, 0)),
                  pl.BlockSpec((1, _H), lambda i: (0, 0)),
                  pl.BlockSpec((1, _H), lambda i: (0, 0)),
                  pl.BlockSpec((_H, 2 * _H), lambda i: (0, 0)),
                  pl.BlockSpec((1, 2 * _H), lambda i: (0, 0))],
        out_specs=[pl.BlockSpec((1, 2 * _H), lambda i: (0, 0)),
                   pl.BlockSpec((1, 2 * _H), lambda i: (0, 0))],
        out_shape=[jax.ShapeDtypeStruct((1, 2 * _H), _F32)] * 2,
    )(x, w1, b1, m1, g1, be1, w2, b2)


def _p3_gate(x, w1, b1, m1, g1, be1, w2, b2, m2, g2, be2):
    def body(x_ref, w1_ref, b1_ref, m1_ref, g1_ref, be1_ref, w2_ref, b2_ref,
             m2_ref, g2_ref, be2_ref, o_ref):
        y1 = _dot(x_ref[...], w1_ref[...]) + b1_ref[...]
        z = _silu((y1 - m1_ref[...]) * g1_ref[...] + be1_ref[...])
        y = _dot(z, w2_ref[...]) + b2_ref[...]
        y = (y - m2_ref[...]) * g2_ref[...] + be2_ref[...]
        u = y[:, :_H]
        v = y[:, _H:]
        o_ref[...] = _sigmoid(u) * _softplus(v)

    return pl.pallas_call(
        body,
        grid=(_E // _TE,),
        in_specs=[pl.BlockSpec((_TE, _CHEM), lambda i: (i, 0)),
                  pl.BlockSpec((_CHEM, _H), lambda i: (0, 0)),
                  pl.BlockSpec((1, _H), lambda i: (0, 0)),
                  pl.BlockSpec((1, _H), lambda i: (0, 0)),
                  pl.BlockSpec((1, _H), lambda i: (0, 0)),
                  pl.BlockSpec((1, _H), lambda i: (0, 0)),
                  pl.BlockSpec((_H, 2 * _H), lambda i: (0, 0)),
                  pl.BlockSpec((1, 2 * _H), lambda i: (0, 0)),
                  pl.BlockSpec((1, 2 * _H), lambda i: (0, 0)),
                  pl.BlockSpec((1, 2 * _H), lambda i: (0, 0)),
                  pl.BlockSpec((1, 2 * _H), lambda i: (0, 0))],
        out_specs=pl.BlockSpec((_TE, _H), lambda i: (i, 0)),
        out_shape=jax.ShapeDtypeStruct((_E, _H), _F32),
    )(x, w1, b1, m1, g1, be1, w2, b2, m2, g2, be2)


def _bn_stats(s, q, g):
    m = s / float(_E)
    var = q / float(_E) - m * m
    return m, g / jnp.sqrt(var + _EPS)


def _segment_sum_sc(gated, ids):
    zeros = jnp.zeros((_ACC_ROWS, _H), _F32)
    mesh = plsc.VectorSubcoreMesh(core_axis_name="c", subcore_axis_name="s")

    @functools.partial(
        pl.kernel,
        out_type=jax.ShapeDtypeStruct((_N, _H), _F32),
        mesh=mesh,
        compiler_params=pltpu.CompilerParams(needs_layout_passes=False),
        scratch_types=[
            pltpu.VMEM((_CHUNK,), jnp.int32),
            pltpu.VMEM((_CHUNK,), jnp.int32),
            pltpu.VMEM((128,), jnp.int32),
            pltpu.VMEM((128,), jnp.int32),
            pltpu.VMEM((128,), jnp.int32),
            pltpu.VMEM((128,), jnp.int32),
            pltpu.VMEM((_CHUNK, _H), _F32),
            pltpu.VMEM_SHARED((_ACC_ROWS, _H), _F32),
            pltpu.SemaphoreType.DMA,
            pltpu.SemaphoreType.DMA,
            pltpu.SemaphoreType.DMA,
        ],
    )
    def seg(vals_hbm, ids_hbm, zeros_hbm, out_hbm,
            ids_a, ids_b, ix0, ix1, ix2, ix3, vals_v, acc,
            sem_i, sem_v, sem_s):
        c = lax.axis_index("c")
        s = lax.axis_index("s")
        ix_refs = (ix0, ix1, ix2, ix3)
        rows = _QUART // _NSUB

        def prefetch(t, buf):
            chunk = jnp.minimum(t * _NSUB + s, _NCHUNK - 1)
            return pltpu.async_copy(
                ids_hbm.at[pl.ds(chunk * _CHUNK, _CHUNK)], buf, sem_i)

        def process(t, buf, base):
            lo = jnp.min(buf[pl.ds(0, 16)])
            hi = jnp.max(buf[pl.ds(_CHUNK - 16, 16)])
            hit = jnp.logical_and(hi >= base, lo < base + _QUART)

            @pl.when(hit)
            def _():
                estart = (t * _NSUB + s) * _CHUNK
                hv = pltpu.async_copy(vals_hbm.at[pl.ds(estart, _CHUNK)],
                                      vals_v, sem_v)
                for r in range(4):
                    for k in range(8):
                        sl = buf[pl.ds(r * 128 + k * 16, 16)]
                        rel = sl - base
                        ok = jnp.logical_and(rel >= 0, rel < _QUART)
                        ix_refs[r][pl.ds(k * 16, 16)] = jnp.where(
                            ok, rel, _DUMMY)
                hv.wait()
                hs = [pltpu.async_copy(vals_v.at[pl.ds(r * 128, 128)],
                                       acc.at[ix_refs[r]], sem_s, add=True)
                      for r in range(4)]
                for h in hs:
                    h.wait()

        for p in range(2):
            base = c * _HALF + p * _QUART
            pltpu.sync_copy(zeros_hbm.at[pl.ds(s * _ZROWS, _ZROWS)],
                            acc.at[pl.ds(s * _ZROWS, _ZROWS)])
            plsc.subcore_barrier()

            prefetch(0, ids_a).wait()

            @pl.loop(0, _CPS // 2)
            def _(u):
                t0 = 2 * u
                hb = prefetch(t0 + 1, ids_b)
                process(t0, ids_a, base)
                hb.wait()
                ha = prefetch(t0 + 2, ids_a)
                process(t0 + 1, ids_b, base)
                ha.wait()

            plsc.subcore_barrier()
            pltpu.sync_copy(acc.at[pl.ds(s * rows, rows)],
                            out_hbm.at[pl.ds(base + s * rows, rows)])
            plsc.subcore_barrier()

    return seg(gated, ids, zeros)


def _p5a_geom(gf, gw1, gb1, gg1, gbb1, gw2, gb2, gg2, gbb2):
    def body(gf_ref, gw1_r, gb1_r, gg1_r, gbb1_r, gw2_r, gb2_r,
             gg2_r, gbb2_r, o_ref):
        y = _dot(gf_ref[...], gw1_r[...]) + gb1_r[...]
        y = _silu(_bn(y, gg1_r[...], gbb1_r[...]))
        o_ref[...] = _bn(_dot(y, gw2_r[...]) + gb2_r[...], gg2_r[...],
                         gbb2_r[...])

    return pl.pallas_call(
        body,
        compiler_params=_VMEM_FULL,
        out_shape=jax.ShapeDtypeStruct((_N, _H // 2), _F32),
    )(gf, gw1, gb1, gg1, gbb1, gw2, gb2, gg2, gbb2)


def _p5a_feat(h_chem, hg, fwa, fwb, fb1, fg1, fbb1, fw2, fb2, fg2, fbb2):
    def body(hc_ref, hg_ref, fwa_r, fwb_r, fb1_r, fg1_r, fbb1_r, fw2_r,
             fb2_r, fg2_r, fbb2_r, o_ref):
        y = _dot(hc_ref[...], fwa_r[...]) + _dot(hg_ref[...], fwb_r[...]) \
            + fb1_r[...]
        y = _silu(_bn(y, fg1_r[...], fbb1_r[...]))
        y = _dot(y, fw2_r[...]) + fb2_r[...]
        o_ref[...] = _bn(y, fg2_r[...], fbb2_r[...])

    return pl.pallas_call(
        body,
        compiler_params=_VMEM_FULL,
        out_shape=jax.ShapeDtypeStruct((_N, _H), _F32),
    )(h_chem, hg, fwa, fwb, fb1, fg1, fbb1, fw2, fb2, fg2, fbb2)


def _p5b_spectral(h, vecs, vinv_t, evb, t, m, sd):
    def body(h_ref, v_ref, vi_ref, e_ref, t_ref, m_ref, s_ref, o_ref):
        hs = _dot(vi_ref[0], h_ref[...])
        e = e_ref[0]
        d = m_ref[...] - e
        band = jnp.exp(-(d * d) / (2.0 * s_ref[...] * s_ref[...]))
        prop = jnp.exp(-e * t_ref[...])
        o_ref[...] = _dot(v_ref[0], band * prop * hs)

    return pl.pallas_call(
        body,
        grid=(_B,),
        in_specs=[pl.BlockSpec((_V, _H), lambda b: (b, 0)),
                  pl.BlockSpec((1, _V, _K), lambda b: (b, 0, 0)),
                  pl.BlockSpec((1, _K, _V), lambda b: (b, 0, 0)),
                  pl.BlockSpec((1, _K, _H), lambda b: (b, 0, 0)),
                  pl.BlockSpec((1, _H), lambda b: (0, 0)),
                  pl.BlockSpec((1, _H), lambda b: (0, 0)),
                  pl.BlockSpec((1, _H), lambda b: (0, 0))],
        out_specs=pl.BlockSpec((_V, _H), lambda b: (b, 0)),
        out_shape=jax.ShapeDtypeStruct((_N, _H), _F32),
        compiler_params=_VMEM_FULL,
    )(h, vecs, vinv_t, evb, t, m, sd)


def _p5c_update(h, hp, wa, wb, b1, g1, bb1, w2, b2, g2, bb2):
    def body(h_ref, hp_ref, wa_r, wb_r, b1_r, g1_r, bb1_r, w2_r, b2_r,
             g2_r, bb2_r, o_ref):
        y = _dot(h_ref[...], wa_r[...]) + _dot(hp_ref[...], wb_r[...]) \
            + b1_r[...]
        y = _silu(_bn(y, g1_r[...], bb1_r[...]))
        y = _dot(y, w2_r[...]) + b2_r[...]
        o_ref[...] = h_ref[...] + _bn(y, g2_r[...], bb2_r[...])

    return pl.pallas_call(
        body,
        compiler_params=_VMEM_FULL,
        out_shape=jax.ShapeDtypeStruct((_N, _H), _F32),
    )(h, hp, wa, wb, b1, g1, bb1, w2, b2, g2, bb2)


def _p5d_head(h, wc, bc, gc, bcb, ws, bs):
    def body(h_ref, wc_r, bc_r, gc_r, bcb_r, ws_r, bs_r, o_ref):
        hm = jnp.mean(h_ref[...].reshape(_B, _V, _H), axis=1)
        y = _dot(hm, wc_r[...]) + bc_r[...]
        y = _silu(_bn(y, gc_r[...], bcb_r[...]))
        o_ref[...] = _dot(y, ws_r[...]) + bs_r[...]

    return pl.pallas_call(
        body,
        compiler_params=_VMEM_FULL,
        out_shape=jax.ShapeDtypeStruct((_B, 7), _F32),
    )(h, wc, bc, gc, bcb, ws, bs)


def _row(x):
    return x.reshape(1, -1)


def kernel(chem_feats, nbr_vids, geom_feats, num_verts, eigs, params):
    del num_verts
    cm = params["chem_mlp"]
    w1, b1 = cm["l1"]["W"], _row(cm["l1"]["b"])
    w2, b2 = cm["l2"]["W"], _row(cm["l2"]["b"])

    s1, q1 = _p1_stats(chem_feats, w1, b1)
    m1, g1 = _bn_stats(s1, q1, _row(cm["bn1"]["g"]))
    be1 = _row(cm["bn1"]["b"])

    s2, q2 = _p2_stats(chem_feats, w1, b1, m1, g1, be1, w2, b2)
    m2, g2 = _bn_stats(s2, q2, _row(cm["bn2"]["g"]))
    be2 = _row(cm["bn2"]["b"])

    gated = _p3_gate(chem_feats, w1, b1, m1, g1, be1, w2, b2, m2, g2, be2)
    h_chem = _segment_sum_sc(gated, nbr_vids)

    gm, fm = params["geom_mlp"], params["feat_mlp"]
    fw1 = fm["l1"]["W"]
    hg = _p5a_geom(
        geom_feats,
        gm["l1"]["W"], _row(gm["l1"]["b"]), _row(gm["bn1"]["g"]),
        _row(gm["bn1"]["b"]), gm["l2"]["W"], _row(gm["l2"]["b"]),
        _row(gm["bn2"]["g"]), _row(gm["bn2"]["b"]))
    h = _p5a_feat(
        h_chem, hg,
        fw1[:_H], fw1[_H:], _row(fm["l1"]["b"]), _row(fm["bn1"]["g"]),
        _row(fm["bn1"]["b"]), fm["l2"]["W"], _row(fm["l2"]["b"]),
        _row(fm["bn2"]["g"]), _row(fm["bn2"]["b"]))

    ev = eigs[:, 0, :]
    vecs = eigs[:, 1:1 + _V, :]
    vinv_t = jnp.transpose(eigs[:, 1 + _V:, :], (0, 2, 1))
    evb = jnp.broadcast_to(ev[:, :, None], (_B, _K, _H))

    for lp in params["prop"]:
        t = _row(jnp.clip(lp["t"], 1e-6, None))
        mn = _row(jnp.clip(lp["mean"], 0.001, 5.0))
        sd = _row(jnp.clip(lp["std"], 1.0, None))
        hp = _p5b_spectral(h, vecs, vinv_t, evb, t, mn, sd)
        mp = lp["mlp"]
        mw1 = mp["l1"]["W"]
        h = _p5c_update(
            h, hp, mw1[:_H], mw1[_H:], _row(mp["l1"]["b"]),
            _row(mp["bn1"]["g"]), _row(mp["bn1"]["b"]), mp["l2"]["W"],
            _row(mp["l2"]["b"]), _row(mp["bn2"]["g"]), _row(mp["bn2"]["b"]))

    cl = params["clas"]
    return _p5d_head(h, cl["l"]["W"], _row(cl["l"]["b"]), _row(cl["bn"]["g"]),
                     _row(cl["bn"]["b"]), params["scorer"]["W"],
                     _row(params["scorer"]["b"]))


# speedup vs baseline: 2.3142x; 1.0165x over previous
import functools

import jax
import jax.numpy as jnp
from jax import lax
from jax.experimental import pallas as pl
from jax.experimental.pallas import tpu as pltpu
from jax.experimental.pallas import tpu_sc as plsc

_B = 8
_V = 2048
_N = _B * _V
_E = 262144
_K = 128
_H = 128
_CHEM = 34
_GEOM = 48
_EPS = 1e-5
_TE = 16384
_F32 = jnp.float32

_NSUB = 16
_HALF = _N // 2
_QUART = _N // 4
_CHUNK = 512
_NCHUNK = _E // _CHUNK
_CPS = _NCHUNK // _NSUB
_ACC_ROWS = 4224
_ZROWS = _ACC_ROWS // _NSUB
_DUMMY = _QUART


def _sigmoid(x):
    return 1.0 / (1.0 + jnp.exp(-x))


def _silu(x):
    return x * _sigmoid(x)


def _softplus(x):
    return jnp.maximum(x, 0.0) + jnp.log(1.0 + jnp.exp(-jnp.abs(x)))


def _bn(x, g, b):
    m = jnp.mean(x, axis=0, keepdims=True)
    v = jnp.mean((x - m) * (x - m), axis=0, keepdims=True)
    return g * (x - m) / jnp.sqrt(v + _EPS) + b


def _dot(a, b):
    return jnp.dot(a, b, preferred_element_type=_F32)


_VMEM_FULL = pltpu.CompilerParams(vmem_limit_bytes=64 * 1024 * 1024)


def _p1_stats(x, w1, b1):
    def body(x_ref, w_ref, b_ref, s_ref, q_ref):
        @pl.when(pl.program_id(0) == 0)
        def _():
            s_ref[...] = jnp.zeros_like(s_ref)
            q_ref[...] = jnp.zeros_like(q_ref)
        y = _dot(x_ref[...], w_ref[...]) + b_ref[...]
        s_ref[...] += jnp.sum(y, axis=0, keepdims=True)
        q_ref[...] += jnp.sum(y * y, axis=0, keepdims=True)

    return pl.pallas_call(
        body,
        grid=(_E // _TE,),
        in_specs=[pl.BlockSpec((_TE, _CHEM), lambda i: (i, 0)),
                  pl.BlockSpec((_CHEM, _H), lambda i: (0, 0)),
                  pl.BlockSpec((1, _H), lambda i: (0, 0))],
        out_specs=[pl.BlockSpec((1, _H), lambda i: (0, 0)),
                   pl.BlockSpec((1, _H), lambda i: (0, 0))],
        out_shape=[jax.ShapeDtypeStruct((1, _H), _F32)] * 2,
        compiler_params=_VMEM_FULL,
    )(x, w1, b1)


def _p2_stats(x, w1, b1, m1, g1, be1, w2, b2):
    def body(x_ref, w1_ref, b1_ref, m1_ref, g1_ref, be1_ref, w2_ref, b2_ref,
             s_ref, q_ref):
        @pl.when(pl.program_id(0) == 0)
        def _():
            s_ref[...] = jnp.zeros_like(s_ref)
            q_ref[...] = jnp.zeros_like(q_ref)
        y1 = _dot(x_ref[...], w1_ref[...]) + b1_ref[...]
        z = _silu((y1 - m1_ref[...]) * g1_ref[...] + be1_ref[...])
        y = _dot(z, w2_ref[...]) + b2_ref[...]
        s_ref[...] += jnp.sum(y, axis=0, keepdims=True)
        q_ref[...] += jnp.sum(y * y, axis=0, keepdims=True)

    return pl.pallas_call(
        body,
        grid=(_E // _TE,),
        in_specs=[pl.BlockSpec((_TE, _CHEM), lambda i: (i, 0)),
                  pl.BlockSpec((_CHEM, _H), lambda i: (0, 0)),
                  pl.BlockSpec((1, _H), lambda i: (0, 0)),
                  pl.BlockSpec((1, _H), lambda i: (0, 0)),
                  pl.BlockSpec((1, _H), lambda i: (0, 0)),
                  pl.BlockSpec((1, _H), lambda i: (0, 0)),
                  pl.BlockSpec((_H, 2 * _H), lambda i: (0, 0)),
                  pl.BlockSpec((1, 2 * _H), lambda i: (0, 0))],
        out_specs=[pl.BlockSpec((1, 2 * _H), lambda i: (0, 0)),
                   pl.BlockSpec((1, 2 * _H), lambda i: (0, 0))],
        out_shape=[jax.ShapeDtypeStruct((1, 2 * _H), _F32)] * 2,
        compiler_params=_VMEM_FULL,
    )(x, w1, b1, m1, g1, be1, w2, b2)


def _p3_gate(x, w1, b1, m1, g1, be1, w2, b2, m2, g2, be2):
    def body(x_ref, w1_ref, b1_ref, m1_ref, g1_ref, be1_ref, w2_ref, b2_ref,
             m2_ref, g2_ref, be2_ref, o_ref):
        y1 = _dot(x_ref[...], w1_ref[...]) + b1_ref[...]
        z = _silu((y1 - m1_ref[...]) * g1_ref[...] + be1_ref[...])
        y = _dot(z, w2_ref[...]) + b2_ref[...]
        y = (y - m2_ref[...]) * g2_ref[...] + be2_ref[...]
        u = y[:, :_H]
        v = y[:, _H:]
        o_ref[...] = _sigmoid(u) * _softplus(v)

    return pl.pallas_call(
        body,
        grid=(_E // _TE,),
        in_specs=[pl.BlockSpec((_TE, _CHEM), lambda i: (i, 0)),
                  pl.BlockSpec((_CHEM, _H), lambda i: (0, 0)),
                  pl.BlockSpec((1, _H), lambda i: (0, 0)),
                  pl.BlockSpec((1, _H), lambda i: (0, 0)),
                  pl.BlockSpec((1, _H), lambda i: (0, 0)),
                  pl.BlockSpec((1, _H), lambda i: (0, 0)),
                  pl.BlockSpec((_H, 2 * _H), lambda i: (0, 0)),
                  pl.BlockSpec((1, 2 * _H), lambda i: (0, 0)),
                  pl.BlockSpec((1, 2 * _H), lambda i: (0, 0)),
                  pl.BlockSpec((1, 2 * _H), lambda i: (0, 0)),
                  pl.BlockSpec((1, 2 * _H), lambda i: (0, 0))],
        out_specs=pl.BlockSpec((_TE, _H), lambda i: (i, 0)),
        out_shape=jax.ShapeDtypeStruct((_E, _H), _F32),
        compiler_params=_VMEM_FULL,
    )(x, w1, b1, m1, g1, be1, w2, b2, m2, g2, be2)


def _bn_stats(s, q, g):
    m = s / float(_E)
    var = q / float(_E) - m * m
    return m, g / jnp.sqrt(var + _EPS)


def _segment_sum_sc(gated, ids):
    zeros = jnp.zeros((_ACC_ROWS, _H), _F32)
    mesh = plsc.VectorSubcoreMesh(core_axis_name="c", subcore_axis_name="s")

    @functools.partial(
        pl.kernel,
        out_type=jax.ShapeDtypeStruct((_N, _H), _F32),
        mesh=mesh,
        compiler_params=pltpu.CompilerParams(needs_layout_passes=False),
        scratch_types=[
            pltpu.VMEM((_CHUNK,), jnp.int32),
            pltpu.VMEM((_CHUNK,), jnp.int32),
            pltpu.VMEM((128,), jnp.int32),
            pltpu.VMEM((128,), jnp.int32),
            pltpu.VMEM((128,), jnp.int32),
            pltpu.VMEM((128,), jnp.int32),
            pltpu.VMEM((_CHUNK, _H), _F32),
            pltpu.VMEM_SHARED((_ACC_ROWS, _H), _F32),
            pltpu.SemaphoreType.DMA,
            pltpu.SemaphoreType.DMA,
            pltpu.SemaphoreType.DMA,
        ],
    )
    def seg(vals_hbm, ids_hbm, zeros_hbm, out_hbm,
            ids_a, ids_b, ix0, ix1, ix2, ix3, vals_v, acc,
            sem_i, sem_v, sem_s):
        c = lax.axis_index("c")
        s = lax.axis_index("s")
        ix_refs = (ix0, ix1, ix2, ix3)
        rows = _QUART // _NSUB

        def prefetch(t, buf):
            chunk = jnp.minimum(t * _NSUB + s, _NCHUNK - 1)
            return pltpu.async_copy(
                ids_hbm.at[pl.ds(chunk * _CHUNK, _CHUNK)], buf, sem_i)

        def process(t, buf, base):
            lo = jnp.min(buf[pl.ds(0, 16)])
            hi = jnp.max(buf[pl.ds(_CHUNK - 16, 16)])
            hit = jnp.logical_and(hi >= base, lo < base + _QUART)

            @pl.when(hit)
            def _():
                estart = (t * _NSUB + s) * _CHUNK
                hv = pltpu.async_copy(vals_hbm.at[pl.ds(estart, _CHUNK)],
                                      vals_v, sem_v)
                for r in range(4):
                    for k in range(8):
                        sl = buf[pl.ds(r * 128 + k * 16, 16)]
                        rel = sl - base
                        ok = jnp.logical_and(rel >= 0, rel < _QUART)
                        ix_refs[r][pl.ds(k * 16, 16)] = jnp.where(
                            ok, rel, _DUMMY)
                hv.wait()
                hs = [pltpu.async_copy(vals_v.at[pl.ds(r * 128, 128)],
                                       acc.at[ix_refs[r]], sem_s, add=True)
                      for r in range(4)]
                for h in hs:
                    h.wait()

        for p in range(2):
            base = c * _HALF + p * _QUART
            pltpu.sync_copy(zeros_hbm.at[pl.ds(s * _ZROWS, _ZROWS)],
                            acc.at[pl.ds(s * _ZROWS, _ZROWS)])
            plsc.subcore_barrier()

            prefetch(0, ids_a).wait()

            @pl.loop(0, _CPS // 2)
            def _(u):
                t0 = 2 * u
                hb = prefetch(t0 + 1, ids_b)
                process(t0, ids_a, base)
                hb.wait()
                ha = prefetch(t0 + 2, ids_a)
                process(t0 + 1, ids_b, base)
                ha.wait()

            plsc.subcore_barrier()
            pltpu.sync_copy(acc.at[pl.ds(s * rows, rows)],
                            out_hbm.at[pl.ds(base + s * rows, rows)])
            plsc.subcore_barrier()

    return seg(gated, ids, zeros)


def _p5a_geom(gf, gw1, gb1, gg1, gbb1, gw2, gb2, gg2, gbb2):
    def body(gf_ref, gw1_r, gb1_r, gg1_r, gbb1_r, gw2_r, gb2_r,
             gg2_r, gbb2_r, o_ref):
        y = _dot(gf_ref[...], gw1_r[...]) + gb1_r[...]
        y = _silu(_bn(y, gg1_r[...], gbb1_r[...]))
        o_ref[...] = _bn(_dot(y, gw2_r[...]) + gb2_r[...], gg2_r[...],
                         gbb2_r[...])

    return pl.pallas_call(
        body,
        compiler_params=_VMEM_FULL,
        out_shape=jax.ShapeDtypeStruct((_N, _H // 2), _F32),
    )(gf, gw1, gb1, gg1, gbb1, gw2, gb2, gg2, gbb2)


def _p5a_feat(h_chem, hg, fwa, fwb, fb1, fg1, fbb1, fw2, fb2, fg2, fbb2):
    def body(hc_ref, hg_ref, fwa_r, fwb_r, fb1_r, fg1_r, fbb1_r, fw2_r,
             fb2_r, fg2_r, fbb2_r, o_ref):
        y = _dot(hc_ref[...], fwa_r[...]) + _dot(hg_ref[...], fwb_r[...]) \
            + fb1_r[...]
        y = _silu(_bn(y, fg1_r[...], fbb1_r[...]))
        y = _dot(y, fw2_r[...]) + fb2_r[...]
        o_ref[...] = _bn(y, fg2_r[...], fbb2_r[...])

    return pl.pallas_call(
        body,
        compiler_params=_VMEM_FULL,
        out_shape=jax.ShapeDtypeStruct((_N, _H), _F32),
    )(h_chem, hg, fwa, fwb, fb1, fg1, fbb1, fw2, fb2, fg2, fbb2)


def _p5b_spectral(h, vecs, vinv_t, evb, t, m, sd):
    def body(h_ref, v_ref, vi_ref, e_ref, t_ref, m_ref, s_ref, o_ref):
        hs = _dot(vi_ref[0], h_ref[...])
        e = e_ref[0]
        d = m_ref[...] - e
        band = jnp.exp(-(d * d) / (2.0 * s_ref[...] * s_ref[...]))
        prop = jnp.exp(-e * t_ref[...])
        o_ref[...] = _dot(v_ref[0], band * prop * hs)

    return pl.pallas_call(
        body,
        grid=(_B,),
        in_specs=[pl.BlockSpec((_V, _H), lambda b: (b, 0)),
                  pl.BlockSpec((1, _V, _K), lambda b: (b, 0, 0)),
                  pl.BlockSpec((1, _K, _V), lambda b: (b, 0, 0)),
                  pl.BlockSpec((1, _K, _H), lambda b: (b, 0, 0)),
                  pl.BlockSpec((1, _H), lambda b: (0, 0)),
                  pl.BlockSpec((1, _H), lambda b: (0, 0)),
                  pl.BlockSpec((1, _H), lambda b: (0, 0))],
        out_specs=pl.BlockSpec((_V, _H), lambda b: (b, 0)),
        out_shape=jax.ShapeDtypeStruct((_N, _H), _F32),
        compiler_params=_VMEM_FULL,
    )(h, vecs, vinv_t, evb, t, m, sd)


def _p5c_update(h, hp, wa, wb, b1, g1, bb1, w2, b2, g2, bb2):
    def body(h_ref, hp_ref, wa_r, wb_r, b1_r, g1_r, bb1_r, w2_r, b2_r,
             g2_r, bb2_r, o_ref):
        y = _dot(h_ref[...], wa_r[...]) + _dot(hp_ref[...], wb_r[...]) \
            + b1_r[...]
        y = _silu(_bn(y, g1_r[...], bb1_r[...]))
        y = _dot(y, w2_r[...]) + b2_r[...]
        o_ref[...] = h_ref[...] + _bn(y, g2_r[...], bb2_r[...])

    return pl.pallas_call(
        body,
        compiler_params=_VMEM_FULL,
        out_shape=jax.ShapeDtypeStruct((_N, _H), _F32),
    )(h, hp, wa, wb, b1, g1, bb1, w2, b2, g2, bb2)


def _p5d_head(h, wc, bc, gc, bcb, ws, bs):
    def body(h_ref, wc_r, bc_r, gc_r, bcb_r, ws_r, bs_r, o_ref):
        hm = jnp.mean(h_ref[...].reshape(_B, _V, _H), axis=1)
        y = _dot(hm, wc_r[...]) + bc_r[...]
        y = _silu(_bn(y, gc_r[...], bcb_r[...]))
        o_ref[...] = _dot(y, ws_r[...]) + bs_r[...]

    return pl.pallas_call(
        body,
        compiler_params=_VMEM_FULL,
        out_shape=jax.ShapeDtypeStruct((_B, 7), _F32),
    )(h, wc, bc, gc, bcb, ws, bs)


def _row(x):
    return x.reshape(1, -1)


def kernel(chem_feats, nbr_vids, geom_feats, num_verts, eigs, params):
    del num_verts
    cm = params["chem_mlp"]
    w1, b1 = cm["l1"]["W"], _row(cm["l1"]["b"])
    w2, b2 = cm["l2"]["W"], _row(cm["l2"]["b"])

    s1, q1 = _p1_stats(chem_feats, w1, b1)
    m1, g1 = _bn_stats(s1, q1, _row(cm["bn1"]["g"]))
    be1 = _row(cm["bn1"]["b"])

    s2, q2 = _p2_stats(chem_feats, w1, b1, m1, g1, be1, w2, b2)
    m2, g2 = _bn_stats(s2, q2, _row(cm["bn2"]["g"]))
    be2 = _row(cm["bn2"]["b"])

    gated = _p3_gate(chem_feats, w1, b1, m1, g1, be1, w2, b2, m2, g2, be2)
    h_chem = _segment_sum_sc(gated, nbr_vids)

    gm, fm = params["geom_mlp"], params["feat_mlp"]
    fw1 = fm["l1"]["W"]
    hg = _p5a_geom(
        geom_feats,
        gm["l1"]["W"], _row(gm["l1"]["b"]), _row(gm["bn1"]["g"]),
        _row(gm["bn1"]["b"]), gm["l2"]["W"], _row(gm["l2"]["b"]),
        _row(gm["bn2"]["g"]), _row(gm["bn2"]["b"]))
    h = _p5a_feat(
        h_chem, hg,
        fw1[:_H], fw1[_H:], _row(fm["l1"]["b"]), _row(fm["bn1"]["g"]),
        _row(fm["bn1"]["b"]), fm["l2"]["W"], _row(fm["l2"]["b"]),
        _row(fm["bn2"]["g"]), _row(fm["bn2"]["b"]))

    ev = eigs[:, 0, :]
    vecs = eigs[:, 1:1 + _V, :]
    vinv_t = jnp.transpose(eigs[:, 1 + _V:, :], (0, 2, 1))
    evb = jnp.broadcast_to(ev[:, :, None], (_B, _K, _H))

    for lp in params["prop"]:
        t = _row(jnp.clip(lp["t"], 1e-6, None))
        mn = _row(jnp.clip(lp["mean"], 0.001, 5.0))
        sd = _row(jnp.clip(lp["std"], 1.0, None))
        hp = _p5b_spectral(h, vecs, vinv_t, evb, t, mn, sd)
        mp = lp["mlp"]
        mw1 = mp["l1"]["W"]
        h = _p5c_update(
            h, hp, mw1[:_H], mw1[_H:], _row(mp["l1"]["b"]),
            _row(mp["bn1"]["g"]), _row(mp["bn1"]["b"]), mp["l2"]["W"],
            _row(mp["l2"]["b"]), _row(mp["bn2"]["g"]), _row(mp["bn2"]["b"]))

    cl = params["clas"]
    return _p5d_head(h, cl["l"]["W"], _row(cl["l"]["b"]), _row(cl["bn"]["g"]),
                     _row(cl["bn"]["b"]), params["scorer"]["W"],
                     _row(params["scorer"]["b"]))


# speedup vs baseline: 2.3573x; 1.0186x over previous
import functools

import jax
import jax.numpy as jnp
from jax import lax
from jax.experimental import pallas as pl
from jax.experimental.pallas import tpu as pltpu
from jax.experimental.pallas import tpu_sc as plsc

_B = 8
_V = 2048
_N = _B * _V
_E = 262144
_K = 128
_H = 128
_CHEM = 34
_GEOM = 48
_EPS = 1e-5
_TE = 16384
_F32 = jnp.float32

_NSUB = 16
_HALF = _N // 2
_QUART = _N // 4
_CHUNK = 512
_NCHUNK = _E // _CHUNK
_CPS = _NCHUNK // _NSUB
_ACC_ROWS = 4224
_ZROWS = _ACC_ROWS // _NSUB
_DUMMY = _QUART


def _sigmoid(x):
    return 0.5 * jnp.tanh(0.5 * x) + 0.5


def _silu(x):
    return x * _sigmoid(x)


def _softplus(x):
    return jnp.maximum(x, 0.0) + jnp.log(1.0 + jnp.exp(-jnp.abs(x)))


def _bn(x, g, b):
    m = jnp.mean(x, axis=0, keepdims=True)
    v = jnp.mean((x - m) * (x - m), axis=0, keepdims=True)
    return g * (x - m) / jnp.sqrt(v + _EPS) + b


def _dot(a, b):
    return jnp.dot(a, b, preferred_element_type=_F32)


_VMEM_FULL = pltpu.CompilerParams(vmem_limit_bytes=64 * 1024 * 1024)


def _p1_stats(x, w1, b1):
    def body(x_ref, w_ref, b_ref, s_ref, q_ref):
        @pl.when(pl.program_id(0) == 0)
        def _():
            s_ref[...] = jnp.zeros_like(s_ref)
            q_ref[...] = jnp.zeros_like(q_ref)
        y = _dot(x_ref[...], w_ref[...]) + b_ref[...]
        s_ref[...] += jnp.sum(y, axis=0, keepdims=True)
        q_ref[...] += jnp.sum(y * y, axis=0, keepdims=True)

    return pl.pallas_call(
        body,
        grid=(_E // _TE,),
        in_specs=[pl.BlockSpec((_TE, _CHEM), lambda i: (i, 0)),
                  pl.BlockSpec((_CHEM, _H), lambda i: (0, 0)),
                  pl.BlockSpec((1, _H), lambda i: (0, 0))],
        out_specs=[pl.BlockSpec((1, _H), lambda i: (0, 0)),
                   pl.BlockSpec((1, _H), lambda i: (0, 0))],
        out_shape=[jax.ShapeDtypeStruct((1, _H), _F32)] * 2,
        compiler_params=_VMEM_FULL,
    )(x, w1, b1)


def _p2_stats(x, w1, b1, m1, g1, be1, w2, b2):
    def body(x_ref, w1_ref, b1_ref, m1_ref, g1_ref, be1_ref, w2_ref, b2_ref,
             s_ref, q_ref):
        @pl.when(pl.program_id(0) == 0)
        def _():
            s_ref[...] = jnp.zeros_like(s_ref)
            q_ref[...] = jnp.zeros_like(q_ref)
        y1 = _dot(x_ref[...], w1_ref[...]) + b1_ref[...]
        z = _silu((y1 - m1_ref[...]) * g1_ref[...] + be1_ref[...])
        y = _dot(z, w2_ref[...]) + b2_ref[...]
        s_ref[...] += jnp.sum(y, axis=0, keepdims=True)
        q_ref[...] += jnp.sum(y * y, axis=0, keepdims=True)

    return pl.pallas_call(
        body,
        grid=(_E // _TE,),
        in_specs=[pl.BlockSpec((_TE, _CHEM), lambda i: (i, 0)),
                  pl.BlockSpec((_CHEM, _H), lambda i: (0, 0)),
                  pl.BlockSpec((1, _H), lambda i: (0, 0)),
                  pl.BlockSpec((1, _H), lambda i: (0, 0)),
                  pl.BlockSpec((1, _H), lambda i: (0, 0)),
                  pl.BlockSpec((1, _H), lambda i: (0, 0)),
                  pl.BlockSpec((_H, 2 * _H), lambda i: (0, 0)),
                  pl.BlockSpec((1, 2 * _H), lambda i: (0, 0))],
        out_specs=[pl.BlockSpec((1, 2 * _H), lambda i: (0, 0)),
                   pl.BlockSpec((1, 2 * _H), lambda i: (0, 0))],
        out_shape=[jax.ShapeDtypeStruct((1, 2 * _H), _F32)] * 2,
        compiler_params=_VMEM_FULL,
    )(x, w1, b1, m1, g1, be1, w2, b2)


def _p3_gate(x, w1, b1, m1, g1, be1, w2, b2, m2, g2, be2):
    def body(x_ref, w1_ref, b1_ref, m1_ref, g1_ref, be1_ref, w2_ref, b2_ref,
             m2_ref, g2_ref, be2_ref, o_ref):
        y1 = _dot(x_ref[...], w1_ref[...]) + b1_ref[...]
        z = _silu((y1 - m1_ref[...]) * g1_ref[...] + be1_ref[...])
        y = _dot(z, w2_ref[...]) + b2_ref[...]
        y = (y - m2_ref[...]) * g2_ref[...] + be2_ref[...]
        u = y[:, :_H]
        v = y[:, _H:]
        o_ref[...] = _sigmoid(u) * _softplus(v)

    return pl.pallas_call(
        body,
        grid=(_E // _TE,),
        in_specs=[pl.BlockSpec((_TE, _CHEM), lambda i: (i, 0)),
                  pl.BlockSpec((_CHEM, _H), lambda i: (0, 0)),
                  pl.BlockSpec((1, _H), lambda i: (0, 0)),
                  pl.BlockSpec((1, _H), lambda i: (0, 0)),
                  pl.BlockSpec((1, _H), lambda i: (0, 0)),
                  pl.BlockSpec((1, _H), lambda i: (0, 0)),
                  pl.BlockSpec((_H, 2 * _H), lambda i: (0, 0)),
                  pl.BlockSpec((1, 2 * _H), lambda i: (0, 0)),
                  pl.BlockSpec((1, 2 * _H), lambda i: (0, 0)),
                  pl.BlockSpec((1, 2 * _H), lambda i: (0, 0)),
                  pl.BlockSpec((1, 2 * _H), lambda i: (0, 0))],
        out_specs=pl.BlockSpec((_TE, _H), lambda i: (i, 0)),
        out_shape=jax.ShapeDtypeStruct((_E, _H), _F32),
        compiler_params=_VMEM_FULL,
    )(x, w1, b1, m1, g1, be1, w2, b2, m2, g2, be2)


def _bn_stats(s, q, g):
    m = s / float(_E)
    var = q / float(_E) - m * m
    return m, g / jnp.sqrt(var + _EPS)


def _segment_sum_sc(gated, ids):
    zeros = jnp.zeros((_ACC_ROWS, _H), _F32)
    mesh = plsc.VectorSubcoreMesh(core_axis_name="c", subcore_axis_name="s")

    @functools.partial(
        pl.kernel,
        out_type=jax.ShapeDtypeStruct((_N, _H), _F32),
        mesh=mesh,
        compiler_params=pltpu.CompilerParams(needs_layout_passes=False),
        scratch_types=[
            pltpu.VMEM((_CHUNK,), jnp.int32),
            pltpu.VMEM((_CHUNK,), jnp.int32),
            pltpu.VMEM((128,), jnp.int32),
            pltpu.VMEM((128,), jnp.int32),
            pltpu.VMEM((128,), jnp.int32),
            pltpu.VMEM((128,), jnp.int32),
            pltpu.VMEM((_CHUNK, _H), _F32),
            pltpu.VMEM_SHARED((_ACC_ROWS, _H), _F32),
            pltpu.SemaphoreType.DMA,
            pltpu.SemaphoreType.DMA,
            pltpu.SemaphoreType.DMA,
        ],
    )
    def seg(vals_hbm, ids_hbm, zeros_hbm, out_hbm,
            ids_a, ids_b, ix0, ix1, ix2, ix3, vals_v, acc,
            sem_i, sem_v, sem_s):
        c = lax.axis_index("c")
        s = lax.axis_index("s")
        ix_refs = (ix0, ix1, ix2, ix3)
        rows = _QUART // _NSUB

        def prefetch(t, buf):
            chunk = jnp.minimum(t * _NSUB + s, _NCHUNK - 1)
            return pltpu.async_copy(
                ids_hbm.at[pl.ds(chunk * _CHUNK, _CHUNK)], buf, sem_i)

        def process(t, buf, base):
            lo = jnp.min(buf[pl.ds(0, 16)])
            hi = jnp.max(buf[pl.ds(_CHUNK - 16, 16)])
            hit = jnp.logical_and(hi >= base, lo < base + _QUART)

            @pl.when(hit)
            def _():
                estart = (t * _NSUB + s) * _CHUNK
                hv = pltpu.async_copy(vals_hbm.at[pl.ds(estart, _CHUNK)],
                                      vals_v, sem_v)
                for r in range(4):
                    for k in range(8):
                        sl = buf[pl.ds(r * 128 + k * 16, 16)]
                        rel = sl - base
                        ok = jnp.logical_and(rel >= 0, rel < _QUART)
                        ix_refs[r][pl.ds(k * 16, 16)] = jnp.where(
                            ok, rel, _DUMMY)
                hv.wait()
                hs = [pltpu.async_copy(vals_v.at[pl.ds(r * 128, 128)],
                                       acc.at[ix_refs[r]], sem_s, add=True)
                      for r in range(4)]
                for h in hs:
                    h.wait()

        for p in range(2):
            base = c * _HALF + p * _QUART
            pltpu.sync_copy(zeros_hbm.at[pl.ds(s * _ZROWS, _ZROWS)],
                            acc.at[pl.ds(s * _ZROWS, _ZROWS)])
            plsc.subcore_barrier()

            prefetch(0, ids_a).wait()

            @pl.loop(0, _CPS // 2)
            def _(u):
                t0 = 2 * u
                hb = prefetch(t0 + 1, ids_b)
                process(t0, ids_a, base)
                hb.wait()
                ha = prefetch(t0 + 2, ids_a)
                process(t0 + 1, ids_b, base)
                ha.wait()

            plsc.subcore_barrier()
            pltpu.sync_copy(acc.at[pl.ds(s * rows, rows)],
                            out_hbm.at[pl.ds(base + s * rows, rows)])
            plsc.subcore_barrier()

    return seg(gated, ids, zeros)


def _p5a_geom(gf, gw1, gb1, gg1, gbb1, gw2, gb2, gg2, gbb2):
    def body(gf_ref, gw1_r, gb1_r, gg1_r, gbb1_r, gw2_r, gb2_r,
             gg2_r, gbb2_r, o_ref):
        y = _dot(gf_ref[...], gw1_r[...]) + gb1_r[...]
        y = _silu(_bn(y, gg1_r[...], gbb1_r[...]))
        o_ref[...] = _bn(_dot(y, gw2_r[...]) + gb2_r[...], gg2_r[...],
                         gbb2_r[...])

    return pl.pallas_call(
        body,
        compiler_params=_VMEM_FULL,
        out_shape=jax.ShapeDtypeStruct((_N, _H // 2), _F32),
    )(gf, gw1, gb1, gg1, gbb1, gw2, gb2, gg2, gbb2)


def _p5a_feat(h_chem, hg, fwa, fwb, fb1, fg1, fbb1, fw2, fb2, fg2, fbb2):
    def body(hc_ref, hg_ref, fwa_r, fwb_r, fb1_r, fg1_r, fbb1_r, fw2_r,
             fb2_r, fg2_r, fbb2_r, o_ref):
        y = _dot(hc_ref[...], fwa_r[...]) + _dot(hg_ref[...], fwb_r[...]) \
            + fb1_r[...]
        y = _silu(_bn(y, fg1_r[...], fbb1_r[...]))
        y = _dot(y, fw2_r[...]) + fb2_r[...]
        o_ref[...] = _bn(y, fg2_r[...], fbb2_r[...])

    return pl.pallas_call(
        body,
        compiler_params=_VMEM_FULL,
        out_shape=jax.ShapeDtypeStruct((_N, _H), _F32),
    )(h_chem, hg, fwa, fwb, fb1, fg1, fbb1, fw2, fb2, fg2, fbb2)


def _p5b_spectral(h, vecs, vinv_t, evb, t, m, sd):
    def body(h_ref, v_ref, vi_ref, e_ref, t_ref, m_ref, s_ref, o_ref):
        hs = _dot(vi_ref[0], h_ref[...])
        e = e_ref[0]
        d = m_ref[...] - e
        band = jnp.exp(-(d * d) / (2.0 * s_ref[...] * s_ref[...]))
        prop = jnp.exp(-e * t_ref[...])
        o_ref[...] = _dot(v_ref[0], band * prop * hs)

    return pl.pallas_call(
        body,
        grid=(_B,),
        in_specs=[pl.BlockSpec((_V, _H), lambda b: (b, 0)),
                  pl.BlockSpec((1, _V, _K), lambda b: (b, 0, 0)),
                  pl.BlockSpec((1, _K, _V), lambda b: (b, 0, 0)),
                  pl.BlockSpec((1, _K, _H), lambda b: (b, 0, 0)),
                  pl.BlockSpec((1, _H), lambda b: (0, 0)),
                  pl.BlockSpec((1, _H), lambda b: (0, 0)),
                  pl.BlockSpec((1, _H), lambda b: (0, 0))],
        out_specs=pl.BlockSpec((_V, _H), lambda b: (b, 0)),
        out_shape=jax.ShapeDtypeStruct((_N, _H), _F32),
        compiler_params=_VMEM_FULL,
    )(h, vecs, vinv_t, evb, t, m, sd)


def _p5c_update(h, hp, wa, wb, b1, g1, bb1, w2, b2, g2, bb2):
    def body(h_ref, hp_ref, wa_r, wb_r, b1_r, g1_r, bb1_r, w2_r, b2_r,
             g2_r, bb2_r, o_ref):
        y = _dot(h_ref[...], wa_r[...]) + _dot(hp_ref[...], wb_r[...]) \
            + b1_r[...]
        y = _silu(_bn(y, g1_r[...], bb1_r[...]))
        y = _dot(y, w2_r[...]) + b2_r[...]
        o_ref[...] = h_ref[...] + _bn(y, g2_r[...], bb2_r[...])

    return pl.pallas_call(
        body,
        compiler_params=_VMEM_FULL,
        out_shape=jax.ShapeDtypeStruct((_N, _H), _F32),
    )(h, hp, wa, wb, b1, g1, bb1, w2, b2, g2, bb2)


def _p5d_head(h, wc, bc, gc, bcb, ws, bs):
    def body(h_ref, wc_r, bc_r, gc_r, bcb_r, ws_r, bs_r, o_ref):
        hm = jnp.mean(h_ref[...].reshape(_B, _V, _H), axis=1)
        y = _dot(hm, wc_r[...]) + bc_r[...]
        y = _silu(_bn(y, gc_r[...], bcb_r[...]))
        o_ref[...] = _dot(y, ws_r[...]) + bs_r[...]

    return pl.pallas_call(
        body,
        compiler_params=_VMEM_FULL,
        out_shape=jax.ShapeDtypeStruct((_B, 7), _F32),
    )(h, wc, bc, gc, bcb, ws, bs)


def _row(x):
    return x.reshape(1, -1)


def kernel(chem_feats, nbr_vids, geom_feats, num_verts, eigs, params):
    del num_verts
    cm = params["chem_mlp"]
    w1, b1 = cm["l1"]["W"], _row(cm["l1"]["b"])
    w2, b2 = cm["l2"]["W"], _row(cm["l2"]["b"])

    s1, q1 = _p1_stats(chem_feats, w1, b1)
    m1, g1 = _bn_stats(s1, q1, _row(cm["bn1"]["g"]))
    be1 = _row(cm["bn1"]["b"])

    s2, q2 = _p2_stats(chem_feats, w1, b1, m1, g1, be1, w2, b2)
    m2, g2 = _bn_stats(s2, q2, _row(cm["bn2"]["g"]))
    be2 = _row(cm["bn2"]["b"])

    gated = _p3_gate(chem_feats, w1, b1, m1, g1, be1, w2, b2, m2, g2, be2)
    h_chem = _segment_sum_sc(gated, nbr_vids)

    gm, fm = params["geom_mlp"], params["feat_mlp"]
    fw1 = fm["l1"]["W"]
    hg = _p5a_geom(
        geom_feats,
        gm["l1"]["W"], _row(gm["l1"]["b"]), _row(gm["bn1"]["g"]),
        _row(gm["bn1"]["b"]), gm["l2"]["W"], _row(gm["l2"]["b"]),
        _row(gm["bn2"]["g"]), _row(gm["bn2"]["b"]))
    h = _p5a_feat(
        h_chem, hg,
        fw1[:_H], fw1[_H:], _row(fm["l1"]["b"]), _row(fm["bn1"]["g"]),
        _row(fm["bn1"]["b"]), fm["l2"]["W"], _row(fm["l2"]["b"]),
        _row(fm["bn2"]["g"]), _row(fm["bn2"]["b"]))

    ev = eigs[:, 0, :]
    vecs = eigs[:, 1:1 + _V, :]
    vinv_t = jnp.transpose(eigs[:, 1 + _V:, :], (0, 2, 1))
    evb = jnp.broadcast_to(ev[:, :, None], (_B, _K, _H))

    for lp in params["prop"]:
        t = _row(jnp.clip(lp["t"], 1e-6, None))
        mn = _row(jnp.clip(lp["mean"], 0.001, 5.0))
        sd = _row(jnp.clip(lp["std"], 1.0, None))
        hp = _p5b_spectral(h, vecs, vinv_t, evb, t, mn, sd)
        mp = lp["mlp"]
        mw1 = mp["l1"]["W"]
        h = _p5c_update(
            h, hp, mw1[:_H], mw1[_H:], _row(mp["l1"]["b"]),
            _row(mp["bn1"]["g"]), _row(mp["bn1"]["b"]), mp["l2"]["W"],
            _row(mp["l2"]["b"]), _row(mp["bn2"]["g"]), _row(mp["bn2"]["b"]))

    cl = params["clas"]
    return _p5d_head(h, cl["l"]["W"], _row(cl["l"]["b"]), _row(cl["bn"]["g"]),
                     _row(cl["bn"]["b"]), params["scorer"]["W"],
                     _row(params["scorer"]["b"]))
